# baseline (speedup 1.0000x reference)
import functools

import jax
import jax.numpy as jnp
from jax import lax
from jax.experimental import pallas as pl
from jax.experimental.pallas import tpu as pltpu
from jax.experimental.pallas import tpu_sc as plsc

B = 4096
F = 26
FP = 32
V = 1000
TOTAL = F * V
D = 16
IN_MLP = F * D
INP = FP * D
H = 400

NC, NS, L = 2, 16, 16
NW = NC * NS
RPW = B // NW
PPW = RPW * FP


def _sc_gather(x4, emb, fcf):
    mesh = plsc.VectorSubcoreMesh(core_axis_name="c", subcore_axis_name="s")

    @functools.partial(
        pl.kernel,
        out_type=[
            jax.ShapeDtypeStruct((NW, PPW, D), jnp.float32),
            jax.ShapeDtypeStruct((NW, RPW, 128), jnp.float32),
        ],
        mesh=mesh,
        compiler_params=pltpu.CompilerParams(
            needs_layout_passes=False, use_tc_tiling_on_sc=False),
        scratch_types=[
            pltpu.VMEM((FP, 128), jnp.int32),
            pltpu.VMEM((PPW, D), jnp.float32),
            pltpu.VMEM((TOTAL,), jnp.float32),
            pltpu.VMEM((RPW, 128), jnp.float32),
            pltpu.SemaphoreType.DMA,
        ],
    )
    def k(x4_hbm, emb_hbm, fcf_hbm, rows_out, fcv_out, idx_v, rows_v, fc_v,
          fcv_v, sem):
        wid = lax.axis_index("s") * NC + lax.axis_index("c")
        pltpu.sync_copy(x4_hbm.at[wid], idx_v)
        cps = [
            pltpu.async_copy(
                emb_hbm.at[idx_v.at[t]],
                rows_v.at[pl.ds(t * 128, 128)],
                sem,
            )
            for t in range(FP)
        ]
        pltpu.sync_copy(fcf_hbm, fc_v)
        iota = lax.iota(jnp.int32, L)

        def body(a, _):
            for kk in range(128 // L):
                xv = idx_v[a, pl.ds(kk * L, L)]
                jp = (kk % 2) * L + iota
                field = jnp.minimum(jp, F - 1)
                val = plsc.load_gather(fc_v, [xv + field * V])
                fcv_v[a * 4 + kk // 2, pl.ds((kk % 2) * L, L)] = val
            return _

        lax.fori_loop(0, FP, body, None)
        for cp in cps:
            cp.wait()
        pltpu.sync_copy(rows_v, rows_out.at[wid])
        pltpu.sync_copy(fcv_v, fcv_out.at[wid])

    return k(x4, emb, fcf)


def _tc_body(embed_ref, fcv_ref, w1, b1, w2, b2, w3, b3, w4, b4, bias,
             out_ref, s_acc):
    phase = pl.program_id(0)
    bt = embed_ref.shape[0] // 4

    @pl.when(phase == 0)
    def _():
        @pl.when(pl.program_id(1) == 0)
        def _():
            s_acc[0] = 0.0

        e = jnp.reshape(embed_ref[...], (bt, INP))[:, :IN_MLP]
        rs = jnp.sum(e, axis=1)
        s_acc[0] += jnp.sum(rs * rs) - jnp.sum(e * e)

    @pl.when(phase == 1)
    def _():
        e = jnp.reshape(embed_ref[...], (bt, INP)).astype(jnp.bfloat16)
        h = jnp.maximum(jnp.dot(e, w1[...], preferred_element_type=jnp.float32)
                        + b1[...], 0.0).astype(jnp.bfloat16)
        h = jnp.maximum(jnp.dot(h, w2[...], preferred_element_type=jnp.float32)
                        + b2[...], 0.0).astype(jnp.bfloat16)
        h = jnp.maximum(jnp.dot(h, w3[...], preferred_element_type=jnp.float32)
                        + b3[...], 0.0)
        mlp = jnp.sum(h * w4[...], axis=1) + b4[0, 0]
        lin = jnp.sum(fcv_ref[...][:, :F], axis=1)
        fm = bias[0, 0] + lin + 0.5 * s_acc[0]
        out_ref[...] = jax.nn.sigmoid(fm + mlp)[:, None]


def _tc_mlp(embed128, fcv128, bias, w1, b1, w2, b2, w3, b3, w4, b4):
    bt = 2048
    nt = B // bt
    full = lambda shape: pl.BlockSpec(shape, lambda p, t: (0, 0))
    return pl.pallas_call(
        _tc_body,
        grid=(2, nt),
        in_specs=[
            pl.BlockSpec((4 * bt, 128), lambda p, t: (t, 0)),
            pl.BlockSpec((bt, 128), lambda p, t: (t, 0)),
            full((INP, H)),
            full((1, H)),
            full((H, H)),
            full((1, H)),
            full((H, H)),
            full((1, H)),
            full((1, H)),
            full((1, 1)),
            full((1, 1)),
        ],
        out_specs=pl.BlockSpec((bt, 1), lambda p, t: (t, 0)),
        out_shape=jax.ShapeDtypeStruct((B, 1), jnp.float32),
        scratch_shapes=[pltpu.SMEM((1,), jnp.float32)],
    )(embed128, fcv128, w1, b1, w2, b2, w3, b3, w4, b4, bias)


def kernel(x, bias, fc, emb, W1, b1, W2, b2, W3, b3, W4, b4):
    xi = x.astype(jnp.int32)
    x4 = jnp.concatenate([xi, xi[:, : FP - F]], axis=1).reshape(NW, FP, 128)
    fcf = fc.reshape(TOTAL)
    rows, fcv = _sc_gather(x4, emb, fcf)
    embed128 = rows.reshape(B * 4, 128)
    fcv128 = fcv.reshape(B, 128)
    w1p = jnp.pad(W1.astype(jnp.bfloat16), ((0, INP - IN_MLP), (0, 0)))
    return _tc_mlp(
        embed128, fcv128, bias.reshape(1, 1),
        w1p, b1.reshape(1, H),
        W2.astype(jnp.bfloat16), b2.reshape(1, H),
        W3.astype(jnp.bfloat16), b3.reshape(1, H),
        W4.reshape(1, H), b4.reshape(1, 1),
    )


# speedup vs baseline: 1.2004x; 1.2004x over previous
import functools

import jax
import jax.numpy as jnp
from jax import lax
from jax.experimental import pallas as pl
from jax.experimental.pallas import tpu as pltpu
from jax.experimental.pallas import tpu_sc as plsc

B = 4096
F = 26
FP = 32
V = 1000
TOTAL = F * V
D = 16
IN_MLP = F * D
INP = FP * D
H = 400

NC, NS, L = 2, 16, 16
NW = NC * NS
RPW = B // NW
PPW = RPW * FP


def _sc_gather(x4, emb1k, fcf):
    mesh = plsc.VectorSubcoreMesh(core_axis_name="c", subcore_axis_name="s")

    @functools.partial(
        pl.kernel,
        out_type=[
            jax.ShapeDtypeStruct((NW, PPW, D), jnp.float32),
            jax.ShapeDtypeStruct((NW, RPW, 128), jnp.float32),
        ],
        mesh=mesh,
        compiler_params=pltpu.CompilerParams(
            needs_layout_passes=False, use_tc_tiling_on_sc=False),
        scratch_types=[
            pltpu.VMEM((FP, 128), jnp.int32),
            pltpu.VMEM((PPW, D), jnp.float32),
            pltpu.VMEM((TOTAL,), jnp.float32),
            pltpu.VMEM((RPW, 128), jnp.float32),
            pltpu.SemaphoreType.DMA,
        ],
    )
    def k(emb_hbm, x4_hbm, fcf_hbm, rows_out, fcv_out, idx_v, rows_v, fc_v,
          fcv_v, sem):
        wid = lax.axis_index("s") * NC + lax.axis_index("c")
        pltpu.sync_copy(x4_hbm.at[wid], idx_v)
        cps = [
            pltpu.async_copy(
                emb_hbm.at[idx_v.at[t]],
                rows_v.at[pl.ds(t * 128, 128)],
                sem,
            )
            for t in range(FP)
        ]
        pltpu.sync_copy(fcf_hbm, fc_v)
        iota = lax.iota(jnp.int32, L)

        def body(a, _):
            for kk in range(128 // L):
                xv = idx_v[a, pl.ds(kk * L, L)]
                jp = (kk % 2) * L + iota
                field = jnp.minimum(jp, F - 1)
                val = plsc.load_gather(fc_v, [xv + field * V])
                fcv_v[a * 4 + kk // 2, pl.ds((kk % 2) * L, L)] = val
            return _

        lax.fori_loop(0, FP, body, None)
        for cp in cps:
            cp.wait()
        pltpu.sync_copy(rows_v, rows_out.at[wid])
        pltpu.sync_copy(fcv_v, fcv_out.at[wid])

    return k(emb1k, x4, fcf)


def _tc_body(embed_ref, fcv_ref, w1, b1, w2, b2, w3, b3, w4, b4, bias,
             out_ref, s_acc, fm_pre):
    i = pl.program_id(0)
    nt = pl.num_programs(0) - 1
    bt = embed_ref.shape[0] // 4

    @pl.when(i == 0)
    def _():
        s_acc[0] = 0.0

    @pl.when(i < nt)
    def _():
        ef = jnp.reshape(embed_ref[...], (bt, INP))
        er = ef[:, :IN_MLP]
        rs = jnp.sum(er, axis=1)
        s_acc[0] += jnp.sum(rs * rs) - jnp.sum(er * er)
        e = ef.astype(jnp.bfloat16)
        h = jnp.maximum(jnp.dot(e, w1[...], preferred_element_type=jnp.float32)
                        + b1[...], 0.0).astype(jnp.bfloat16)
        h = jnp.maximum(jnp.dot(h, w2[...], preferred_element_type=jnp.float32)
                        + b2[...], 0.0).astype(jnp.bfloat16)
        h = jnp.maximum(jnp.dot(h, w3[...], preferred_element_type=jnp.float32)
                        + b3[...], 0.0)
        mlp = jnp.sum(h * w4[...], axis=1) + b4[0, 0]
        lin = jnp.sum(fcv_ref[...][:, :F], axis=1)
        fm_pre[pl.ds(i * bt, bt)] = lin + mlp

    @pl.when(i == nt)
    def _():
        fm = fm_pre[...] + (bias[0, 0] + 0.5 * s_acc[0])
        out_ref[...] = jax.nn.sigmoid(fm)[:, None]


def _tc_mlp(embed128, fcv128, bias, w1, b1, w2, b2, w3, b3, w4, b4):
    bt = 2048
    nt = B // bt
    full = lambda shape: pl.BlockSpec(shape, lambda i: (0, 0))
    tile = lambda i: (jnp.minimum(i, nt - 1), 0)
    return pl.pallas_call(
        _tc_body,
        grid=(nt + 1,),
        in_specs=[
            pl.BlockSpec((4 * bt, 128), tile),
            pl.BlockSpec((bt, 128), tile),
            full((INP, H)),
            full((1, H)),
            full((H, H)),
            full((1, H)),
            full((H, H)),
            full((1, H)),
            full((1, H)),
            full((1, 1)),
            full((1, 1)),
        ],
        out_specs=pl.BlockSpec((B, 1), lambda i: (0, 0)),
        out_shape=jax.ShapeDtypeStruct((B, 1), jnp.float32),
        scratch_shapes=[pltpu.SMEM((1,), jnp.float32),
                        pltpu.VMEM((B,), jnp.float32)],
    )(embed128, fcv128, w1, b1, w2, b2, w3, b3, w4, b4, bias)

def kernel(x, bias, fc, emb, W1, b1, W2, b2, W3, b3, W4, b4):
    xi = x.astype(jnp.int32)
    x4 = jnp.concatenate([xi, xi[:, : FP - F]], axis=1).reshape(NW, FP, 128)
    fcf = fc.reshape(TOTAL)
    rows, fcv = _sc_gather(x4, emb[:V], fcf)
    embed128 = rows.reshape(B * 4, 128)
    fcv128 = fcv.reshape(B, 128)
    w1p = jnp.pad(W1.astype(jnp.bfloat16), ((0, INP - IN_MLP), (0, 0)))
    return _tc_mlp(
        embed128, fcv128, bias.reshape(1, 1),
        w1p, b1.reshape(1, H),
        W2.astype(jnp.bfloat16), b2.reshape(1, H),
        W3.astype(jnp.bfloat16), b3.reshape(1, H),
        W4.reshape(1, H), b4.reshape(1, 1),
    )


# speedup vs baseline: 1.3015x; 1.0842x over previous
import functools

import jax
import jax.numpy as jnp
from jax import lax
from jax.experimental import pallas as pl
from jax.experimental.pallas import tpu as pltpu
from jax.experimental.pallas import tpu_sc as plsc

B = 4096
F = 26
FP = 32
V = 1000
TOTAL = F * V
D = 16
IN_MLP = F * D
INP = FP * D
H = 400

NC, NS, L = 2, 16, 16
NW = NC * NS
RPW = B // NW
PPW = RPW * FP


def _sc_gather(x4, emb1k, fcf):
    mesh = plsc.VectorSubcoreMesh(core_axis_name="c", subcore_axis_name="s")

    @functools.partial(
        pl.kernel,
        out_type=[
            jax.ShapeDtypeStruct((NW, PPW, D), jnp.float32),
            jax.ShapeDtypeStruct((NW, RPW, 128), jnp.float32),
        ],
        mesh=mesh,
        compiler_params=pltpu.CompilerParams(
            needs_layout_passes=False, use_tc_tiling_on_sc=False),
        scratch_types=[
            pltpu.VMEM((FP, 128), jnp.int32),
            pltpu.VMEM((PPW, D), jnp.float32),
            pltpu.VMEM((TOTAL,), jnp.float32),
            pltpu.VMEM((RPW, 128), jnp.float32),
            pltpu.SemaphoreType.DMA,
        ],
    )
    def k(emb_hbm, x4_hbm, fcf_hbm, rows_out, fcv_out, idx_v, rows_v, fc_v,
          fcv_v, sem):
        wid = lax.axis_index("s") * NC + lax.axis_index("c")
        pltpu.sync_copy(x4_hbm.at[wid], idx_v)
        cps = [
            pltpu.async_copy(
                emb_hbm.at[idx_v.at[t]],
                rows_v.at[pl.ds(t * 128, 128)],
                sem,
            )
            for t in range(FP)
        ]
        pltpu.sync_copy(fcf_hbm, fc_v)
        iota = lax.iota(jnp.int32, L)

        def body(a, _):
            for kk in range(128 // L):
                xv = idx_v[a, pl.ds(kk * L, L)]
                jp = (kk % 2) * L + iota
                field = jnp.minimum(jp, F - 1)
                val = plsc.load_gather(fc_v, [xv + field * V])
                fcv_v[a * 4 + kk // 2, pl.ds((kk % 2) * L, L)] = val
            return _

        lax.fori_loop(0, FP, body, None)
        for cp in cps:
            cp.wait()
        pltpu.sync_copy(rows_v, rows_out.at[wid])
        pltpu.sync_copy(fcv_v, fcv_out.at[wid])

    return k(emb1k, x4, fcf)


def _tc_body(embed_ref, fcv_ref, w1, b1, w2, b2, w3, b3, w4, b4, bias,
             out_ref, s_acc, fm_pre):
    i = pl.program_id(0)
    nt = pl.num_programs(0) - 1
    bt = embed_ref.shape[0] // 4

    @pl.when(i == 0)
    def _():
        s_acc[0] = 0.0

    @pl.when(i < nt)
    def _():
        ef = jnp.reshape(embed_ref[...], (bt, INP))
        er = ef[:, :IN_MLP]
        rs = jnp.sum(er, axis=1)
        s_acc[0] += jnp.sum(rs * rs) - jnp.sum(er * er)
        e = ef.astype(jnp.bfloat16)
        h = jnp.maximum(jnp.dot(e, w1[...], preferred_element_type=jnp.float32)
                        + b1[...], 0.0).astype(jnp.bfloat16)
        h = jnp.maximum(jnp.dot(h, w2[...], preferred_element_type=jnp.float32)
                        + b2[...], 0.0).astype(jnp.bfloat16)
        h = jnp.maximum(jnp.dot(h, w3[...], preferred_element_type=jnp.float32)
                        + b3[...], 0.0)
        mlp = jnp.sum(h * w4[...], axis=1) + b4[0, 0]
        lin = jnp.sum(fcv_ref[...][:, :F], axis=1)
        fm_pre[pl.ds(i * bt, bt)] = lin + mlp

    @pl.when(i == nt)
    def _():
        fm = fm_pre[...] + (bias[0, 0] + 0.5 * s_acc[0])
        out_ref[...] = jnp.reshape(jax.nn.sigmoid(fm), (NW, 128))


def _tc_mlp(embed128, fcv128, bias, w1, b1, w2, b2, w3, b3, w4, b4):
    bt = 2048
    nt = B // bt
    full = lambda shape: pl.BlockSpec(shape, lambda i: (0, 0))
    tile = lambda i: (jnp.minimum(i, nt - 1), 0)
    return pl.pallas_call(
        _tc_body,
        grid=(nt + 1,),
        in_specs=[
            pl.BlockSpec((4 * bt, 128), tile),
            pl.BlockSpec((bt, 128), tile),
            full((INP, H)),
            full((1, H)),
            full((H, H)),
            full((1, H)),
            full((H, H)),
            full((1, H)),
            full((1, H)),
            full((1, 1)),
            full((1, 1)),
        ],
        out_specs=pl.BlockSpec((NW, 128), lambda i: (0, 0)),
        out_shape=jax.ShapeDtypeStruct((NW, 128), jnp.float32),
        scratch_shapes=[pltpu.SMEM((1,), jnp.float32),
                        pltpu.VMEM((B,), jnp.float32)],
    )(embed128, fcv128, w1, b1, w2, b2, w3, b3, w4, b4, bias)

def kernel(x, bias, fc, emb, W1, b1, W2, b2, W3, b3, W4, b4):
    xi = x.astype(jnp.int32)
    x4 = jnp.concatenate([xi, xi[:, : FP - F]], axis=1).reshape(NW, FP, 128)
    fcf = fc.reshape(TOTAL)
    rows, fcv = _sc_gather(x4, emb[:V], fcf)
    embed128 = rows.reshape(B * 4, 128)
    fcv128 = fcv.reshape(B, 128)
    w1p = jnp.pad(W1.astype(jnp.bfloat16), ((0, INP - IN_MLP), (0, 0)))
    out = _tc_mlp(
        embed128, fcv128, bias.reshape(1, 1),
        w1p, b1.reshape(1, H),
        W2.astype(jnp.bfloat16), b2.reshape(1, H),
        W3.astype(jnp.bfloat16), b3.reshape(1, H),
        W4.reshape(1, H), b4.reshape(1, 1),
    )
    return out.reshape(B, 1)


# speedup vs baseline: 1.3395x; 1.0292x over previous
import functools

import jax
import jax.numpy as jnp
from jax import lax
from jax.experimental import pallas as pl
from jax.experimental.pallas import tpu as pltpu
from jax.experimental.pallas import tpu_sc as plsc

B = 4096
F = 26
FP = 32
V = 1000
TOTAL = F * V
D = 16
IN_MLP = F * D
INP = FP * D
H = 400

NC, NS, L = 2, 16, 16
NW = NC * NS
RPW = B // NW
PPW = RPW * FP


def _sc_gather(x4, emb1k, fcf):
    mesh = plsc.VectorSubcoreMesh(core_axis_name="c", subcore_axis_name="s")

    @functools.partial(
        pl.kernel,
        out_type=[
            jax.ShapeDtypeStruct((NW, PPW, D), jnp.float32),
            jax.ShapeDtypeStruct((NW, RPW, 128), jnp.float32),
        ],
        mesh=mesh,
        compiler_params=pltpu.CompilerParams(
            needs_layout_passes=False, use_tc_tiling_on_sc=False),
        scratch_types=[
            pltpu.VMEM((FP, 128), jnp.int32),
            pltpu.VMEM((PPW, D), jnp.float32),
            pltpu.VMEM((TOTAL,), jnp.float32),
            pltpu.VMEM((RPW, 128), jnp.float32),
            pltpu.SemaphoreType.DMA,
        ],
    )
    def k(emb_hbm, x4_hbm, fcf_hbm, rows_out, fcv_out, idx_v, rows_v, fc_v,
          fcv_v, sem):
        wid = lax.axis_index("s") * NC + lax.axis_index("c")
        pltpu.sync_copy(x4_hbm.at[wid], idx_v)
        cps = [
            pltpu.async_copy(
                emb_hbm.at[idx_v.at[t]],
                rows_v.at[pl.ds(t * 128, 128)],
                sem,
            )
            for t in range(FP)
        ]
        pltpu.sync_copy(fcf_hbm, fc_v)
        iota = lax.iota(jnp.int32, L)

        def body(a, _):
            for kk in range(128 // L):
                xv = idx_v[a, pl.ds(kk * L, L)]
                jp = (kk % 2) * L + iota
                field = jnp.minimum(jp, F - 1)
                val = plsc.load_gather(fc_v, [xv + field * V])
                fcv_v[a * 4 + kk // 2, pl.ds((kk % 2) * L, L)] = val
            return _

        lax.fori_loop(0, FP, body, None)
        for cp in cps:
            cp.wait()
        pltpu.sync_copy(rows_v, rows_out.at[wid])
        pltpu.sync_copy(fcv_v, fcv_out.at[wid])

    return k(emb1k, x4, fcf)


def _tc_body(embed_ref, fcv_ref, w1, b1, w2, b2, w3, b3, w4, b4, bias,
             out_ref, s_acc, fm_pre):
    i = pl.program_id(0)
    nt = pl.num_programs(0) - 1
    bt = embed_ref.shape[0] // 4

    @pl.when(i == 0)
    def _():
        s_acc[0] = 0.0

    @pl.when(i < nt)
    def _():
        ef = jnp.reshape(embed_ref[...], (bt, INP))
        er = ef[:, :IN_MLP]
        rs = jnp.sum(er, axis=1)
        s_acc[0] += jnp.sum(rs * rs) - jnp.sum(er * er)
        e = ef.astype(jnp.bfloat16)
        h = jnp.maximum(jnp.dot(e, w1[...], preferred_element_type=jnp.float32)
                        + b1[...], 0.0).astype(jnp.bfloat16)
        h = jnp.maximum(jnp.dot(h, w2[...], preferred_element_type=jnp.float32)
                        + b2[...], 0.0).astype(jnp.bfloat16)
        h = jnp.maximum(jnp.dot(h, w3[...], preferred_element_type=jnp.float32)
                        + b3[...], 0.0)
        mlp = jnp.sum(h * w4[...], axis=1) + b4[0, 0]
        lin = jnp.sum(fcv_ref[...][:, :F], axis=1)
        fm_pre[pl.ds(i * bt, bt)] = lin + mlp

    @pl.when(i == nt)
    def _():
        fm = fm_pre[...] + (bias[0, 0] + 0.5 * s_acc[0])
        out_ref[...] = jnp.reshape(jax.nn.sigmoid(fm), (NW, 128))


def _tc_mlp(embed128, fcv128, bias, w1, b1, w2, b2, w3, b3, w4, b4):
    bt = 1024
    nt = B // bt
    full = lambda shape: pl.BlockSpec(shape, lambda i: (0, 0))
    tile = lambda i: (jnp.minimum(i, nt - 1), 0)
    return pl.pallas_call(
        _tc_body,
        grid=(nt + 1,),
        in_specs=[
            pl.BlockSpec((4 * bt, 128), tile),
            pl.BlockSpec((bt, 128), tile),
            full((INP, H)),
            full((1, H)),
            full((H, H)),
            full((1, H)),
            full((H, H)),
            full((1, H)),
            full((1, H)),
            full((1, 1)),
            full((1, 1)),
        ],
        out_specs=pl.BlockSpec((NW, 128), lambda i: (0, 0)),
        out_shape=jax.ShapeDtypeStruct((NW, 128), jnp.float32),
        scratch_shapes=[pltpu.SMEM((1,), jnp.float32),
                        pltpu.VMEM((B,), jnp.float32)],
    )(embed128, fcv128, w1, b1, w2, b2, w3, b3, w4, b4, bias)

def kernel(x, bias, fc, emb, W1, b1, W2, b2, W3, b3, W4, b4):
    xi = x.astype(jnp.int32)
    x4 = jnp.concatenate([xi, xi[:, : FP - F]], axis=1).reshape(NW, FP, 128)
    fcf = fc.reshape(TOTAL)
    rows, fcv = _sc_gather(x4, emb[:V], fcf)
    embed128 = rows.reshape(B * 4, 128)
    fcv128 = fcv.reshape(B, 128)
    w1p = jnp.pad(W1.astype(jnp.bfloat16), ((0, INP - IN_MLP), (0, 0)))
    out = _tc_mlp(
        embed128, fcv128, bias.reshape(1, 1),
        w1p, b1.reshape(1, H),
        W2.astype(jnp.bfloat16), b2.reshape(1, H),
        W3.astype(jnp.bfloat16), b3.reshape(1, H),
        W4.reshape(1, H), b4.reshape(1, 1),
    )
    return out.reshape(B, 1)
